# Initial kernel scaffold; baseline (speedup 1.0000x reference)
import jax
import jax.numpy as jnp
from jax.experimental import pallas as pl


def kernel(box_cls, box_regression, iou_pred, anchors):
    raise NotImplementedError("write your pallas kernel here")


# speedup vs baseline: 1.0134x; 1.0134x over previous
import math

import jax
import jax.numpy as jnp
from jax.experimental import pallas as pl
from jax.experimental.pallas import tpu as pltpu

_THRESH = 0.05
_TOP_N = 1000
_NMS_T = 0.6
_POST_N = 100
_IMG_W = 1024.0
_IMG_H = 1024.0
_WX, _WY, _WW, _WH = 10.0, 10.0, 5.0, 5.0
_CLIP = math.log(1000.0 / 16.0)
_PAD = 1024


def _sigmoid(x):
    return 1.0 / (1.0 + jnp.exp(-x))


def _score_body(cls_ref, iou_ref, out_ref):
    s = jnp.sqrt(_sigmoid(cls_ref[...]) * _sigmoid(iou_ref[0]))
    out_ref[...] = jnp.where(s > _THRESH, s, 0.0)


def _scores(box_cls, iou_pred, N, A, C, P):
    cls2 = box_cls.reshape(N * A * C, P)
    iou2 = iou_pred.reshape(N * A, 1, P)
    return pl.pallas_call(
        _score_body,
        grid=(N * A,),
        in_specs=[
            pl.BlockSpec((C, P), lambda i: (i, 0)),
            pl.BlockSpec((1, 1, P), lambda i: (i, 0, 0)),
        ],
        out_specs=pl.BlockSpec((C, P), lambda i: (i, 0)),
        out_shape=jax.ShapeDtypeStruct((N * A * C, P), jnp.float32),
    )(cls2, iou2)


def _nms_body(bx_ref, sc_ref, out_ref):
    x1 = bx_ref[0, 0, :]
    y1 = bx_ref[0, 1, :]
    x2 = bx_ref[0, 2, :]
    y2 = bx_ref[0, 3, :]
    sc = sc_ref[0, 0, :]
    area = jnp.clip(x2 - x1 + 1.0, 0.0, None) * jnp.clip(y2 - y1 + 1.0, 0.0, None)
    nchunk = _PAD // 128
    for r in range(nchunk):
        sl = slice(r * 128, (r + 1) * 128)
        rx1 = x1[sl][:, None]
        ry1 = y1[sl][:, None]
        rx2 = x2[sl][:, None]
        ry2 = y2[sl][:, None]
        rsc = sc[sl][:, None]
        rarea = area[sl][:, None]
        w = jnp.clip(jnp.minimum(rx2, x2[None, :]) - jnp.maximum(rx1, x1[None, :]) + 1.0, 0.0, None)
        h = jnp.clip(jnp.minimum(ry2, y2[None, :]) - jnp.maximum(ry1, y1[None, :]) + 1.0, 0.0, None)
        inter = w * h
        union = rarea + area[None, :] - inter
        iou = inter / jnp.maximum(union, 1e-6)
        hi = (sc[None, :] > rsc) & (iou > _NMS_T)
        sup = jnp.any(hi, axis=1)
        out_ref[0, 0, sl] = jnp.where(sup, 0.0, sc[sl])


def _nms(shifted_t, sc_p, N):
    return pl.pallas_call(
        _nms_body,
        grid=(N,),
        in_specs=[
            pl.BlockSpec((1, 4, _PAD), lambda n: (n, 0, 0)),
            pl.BlockSpec((1, 1, _PAD), lambda n: (n, 0, 0)),
        ],
        out_specs=pl.BlockSpec((1, 1, _PAD), lambda n: (n, 0, 0)),
        out_shape=jax.ShapeDtypeStruct((N, 1, _PAD), jnp.float32),
    )(shifted_t, sc_p)


def _decode(rel, anc):
    TO_REMOVE = 1.0
    widths = anc[..., 2] - anc[..., 0] + TO_REMOVE
    heights = anc[..., 3] - anc[..., 1] + TO_REMOVE
    ctr_x = anc[..., 0] + 0.5 * widths
    ctr_y = anc[..., 1] + 0.5 * heights
    dx = rel[..., 0] / _WX
    dy = rel[..., 1] / _WY
    dw = jnp.minimum(rel[..., 2] / _WW, _CLIP)
    dh = jnp.minimum(rel[..., 3] / _WH, _CLIP)
    pred_ctr_x = dx * widths + ctr_x
    pred_ctr_y = dy * heights + ctr_y
    pred_w = jnp.exp(dw) * widths
    pred_h = jnp.exp(dh) * heights
    return jnp.stack(
        [
            pred_ctr_x - 0.5 * pred_w,
            pred_ctr_y - 0.5 * pred_h,
            pred_ctr_x + 0.5 * pred_w - 1.0,
            pred_ctr_y + 0.5 * pred_h - 1.0,
        ],
        axis=-1,
    )


def kernel(box_cls, box_regression, iou_pred, anchors):
    N, AC, H, W = box_cls.shape
    A = box_regression.shape[1] // 4
    C = AC // A
    P = H * W

    scores_acp = _scores(box_cls, iou_pred, N, A, C, P)

    s = scores_acp.reshape(N, A, C, P).transpose(0, 3, 1, 2).reshape(N, P * A * C)
    ts, ti = jax.lax.top_k(s, _TOP_N)
    loc = ti // C
    labels = ti % C + 1

    reg = box_regression.reshape(N, A, 4, P).transpose(0, 3, 1, 2).reshape(N, P * A, 4)
    reg_sel = jnp.take_along_axis(reg, loc[..., None], axis=1)
    anc_sel = jnp.take_along_axis(anchors, loc[..., None], axis=1)

    boxes = _decode(reg_sel, anc_sel)
    boxes = jnp.stack(
        [
            jnp.clip(boxes[..., 0], 0.0, _IMG_W - 1.0),
            jnp.clip(boxes[..., 1], 0.0, _IMG_H - 1.0),
            jnp.clip(boxes[..., 2], 0.0, _IMG_W - 1.0),
            jnp.clip(boxes[..., 3], 0.0, _IMG_H - 1.0),
        ],
        axis=-1,
    )
    ws = boxes[..., 2] - boxes[..., 0] + 1.0
    hs = boxes[..., 3] - boxes[..., 1] + 1.0
    keep = (ws >= 0.0) & (hs >= 0.0) & (ts > 0.0)
    sc = jnp.where(keep, ts, 0.0)

    off = labels.astype(jnp.float32) * (_IMG_W + _IMG_H)
    shifted = boxes + off[..., None]

    pad = _PAD - _TOP_N
    shifted_t = jnp.pad(shifted.transpose(0, 2, 1), ((0, 0), (0, 0), (0, pad)))
    sc_p = jnp.pad(sc, ((0, 0), (0, pad)))[:, None, :]
    final_sc = _nms(shifted_t, sc_p, N)[:, 0, :_TOP_N]

    fs, fi = jax.lax.top_k(final_sc, _POST_N)
    fb = jnp.take_along_axis(boxes, fi[..., None], axis=1)
    fl = jnp.take_along_axis(labels, fi, axis=1)
    return jnp.concatenate([fb, fs[..., None], fl.astype(jnp.float32)[..., None]], axis=-1)
